# baseline (speedup 1.0000x reference)
import functools

import jax
import jax.numpy as jnp
from jax.experimental import pallas as pl
from jax.experimental.pallas import tpu as pltpu

_N = 20000
_NPAD = 20480
_ROWS = 160
_NUM_CLASSES = 8
_K = 500
_KPAD = 512
_SCORE_THRESHOLD = 0.01
_NMS_THRESHOLD = 0.5
_MAX_DET = 100
_NEG = -3.0e38
_BIGI = 2**30


def _phase1_body(cls_ref, boxes_ref, scores_out_ref, idx_out_ref,
                 x_scr, s_scr, iou_scr):
    x0 = jnp.concatenate(
        [cls_ref[0], jnp.full((32, 128), _NEG, jnp.float32)], axis=0)
    x_scr[...] = x0
    s_scr[...] = jnp.zeros((_KPAD, 128), jnp.float32)

    li = jax.lax.broadcasted_iota(jnp.int32, (1, 128), 1)
    ri24 = jax.lax.broadcasted_iota(jnp.int32, (24, 128), 0)
    ri8 = jax.lax.broadcasted_iota(jnp.int32, (8, 128), 0)
    li8 = jax.lax.broadcasted_iota(jnp.int32, (8, 128), 1)

    partials0 = jnp.max(x0.reshape(24, 8, 128), axis=1)

    def extract(t, partials):
        m = jnp.max(partials)
        b = jnp.min(jnp.where(partials == m, ri24, _BIGI))
        xb = x_scr[pl.ds(b * 8, 8), :]
        keyb = jnp.where(xb == m, (ri8 + b * 8) * 128 + li8, _BIGI)
        f = jnp.min(keyb)
        r = f >> 7
        l = f & 127
        xrow = x_scr[pl.ds(r, 1), :]
        x_scr[pl.ds(r, 1), :] = jnp.where(li == l, _NEG, xrow)
        nb = jnp.max(jnp.where(keyb == f, _NEG, xb), axis=0, keepdims=True)
        partials = jnp.where(ri24 == b, jnp.broadcast_to(nb, (24, 128)),
                             partials)
        lane_m = (li == l)
        c0 = jnp.sum(jnp.where(lane_m, boxes_ref[0, pl.ds(r, 1), :], 0.0))
        c1 = jnp.sum(jnp.where(lane_m, boxes_ref[1, pl.ds(r, 1), :], 0.0))
        c2 = jnp.sum(jnp.where(lane_m, boxes_ref[2, pl.ds(r, 1), :], 0.0))
        c3 = jnp.sum(jnp.where(lane_m, boxes_ref[3, pl.ds(r, 1), :], 0.0))
        svec = jnp.where(li == 0, c0,
               jnp.where(li == 1, c1,
               jnp.where(li == 2, c2,
               jnp.where(li == 3, c3,
               jnp.where(li == 4, m,
               jnp.where(li == 5, f.astype(jnp.float32), 0.0))))))
        s_scr[pl.ds(t, 1), :] = svec
        return partials

    jax.lax.fori_loop(0, _K, extract, partials0)

    s = s_scr[...]
    st = s.T
    x1r = st[0:1, :]
    y1r = st[1:2, :]
    x2r = st[2:3, :]
    y2r = st[3:4, :]
    vals = st[4:5, :]
    idxr = st[5:6, :]
    x1c = s[:, 0:1]
    y1c = s[:, 1:2]
    x2c = s[:, 2:3]
    y2c = s[:, 3:4]

    area_r = jnp.maximum(x2r - x1r, 0.0) * jnp.maximum(y2r - y1r, 0.0)
    area_c = jnp.maximum(x2c - x1c, 0.0) * jnp.maximum(y2c - y1c, 0.0)
    sh = (_KPAD, _KPAD)
    xx1 = jnp.maximum(jnp.broadcast_to(x1c, sh), jnp.broadcast_to(x1r, sh))
    yy1 = jnp.maximum(jnp.broadcast_to(y1c, sh), jnp.broadcast_to(y1r, sh))
    xx2 = jnp.minimum(jnp.broadcast_to(x2c, sh), jnp.broadcast_to(x2r, sh))
    yy2 = jnp.minimum(jnp.broadcast_to(y2c, sh), jnp.broadcast_to(y2r, sh))
    inter = jnp.maximum(xx2 - xx1, 0.0) * jnp.maximum(yy2 - yy1, 0.0)
    union = jnp.broadcast_to(area_c, sh) + jnp.broadcast_to(area_r, sh) - inter
    iou_scr[...] = inter / jnp.maximum(union, 1e-8)

    li512 = jax.lax.broadcasted_iota(jnp.int32, (1, _KPAD), 1)
    keep0 = jnp.where(vals > _SCORE_THRESHOLD, 1.0, 0.0)

    def nms_step(i, keep):
        row = iou_scr[pl.ds(i, 1), :]
        keep_i = jnp.max(jnp.where(li512 == i, keep, 0.0))
        suppress = (row > _NMS_THRESHOLD) & (li512 > i) & (keep_i > 0.5)
        return jnp.where(suppress, 0.0, keep)

    keep = jax.lax.fori_loop(0, _K, nms_step, keep0)

    scores_out_ref[0] = jnp.where(keep > 0.5, vals, -1.0)
    idx_out_ref[0] = idxr.astype(jnp.int32)


def _phase2_body(sc_ref, ix_ref, boxes_ref, rot_ref, trans_ref,
                 b_out, s_out, l_out, r_out, t_out, y_scr):
    y_scr[...] = sc_ref[...]
    li = jax.lax.broadcasted_iota(jnp.int32, (1, 128), 1)
    flatiota = jax.lax.broadcasted_iota(jnp.int32, (32, 128), 0) * 128 + \
        jax.lax.broadcasted_iota(jnp.int32, (32, 128), 1)

    def pick(t, carry):
        y = y_scr[...]
        m = jnp.max(y)
        p = jnp.min(jnp.where(y == m, flatiota, _BIGI))
        pr = p >> 7
        pln = p & 127
        yrow = y_scr[pl.ds(pr, 1), :]
        y_scr[pl.ds(pr, 1), :] = jnp.where(li == pln, _NEG, yrow)
        cls = p >> 9
        f = jnp.sum(jnp.where(li == pln, ix_ref[pl.ds(pr, 1), :], 0.0)
                    ).astype(jnp.int32)
        valid = m > -0.5
        rr = f >> 7
        rl = f & 127
        lane_m = (li == rl)
        b0 = jnp.sum(jnp.where(lane_m, boxes_ref[0, pl.ds(rr, 1), :], 0.0))
        b1 = jnp.sum(jnp.where(lane_m, boxes_ref[1, pl.ds(rr, 1), :], 0.0))
        b2 = jnp.sum(jnp.where(lane_m, boxes_ref[2, pl.ds(rr, 1), :], 0.0))
        b3 = jnp.sum(jnp.where(lane_m, boxes_ref[3, pl.ds(rr, 1), :], 0.0))
        q0 = jnp.sum(jnp.where(lane_m, rot_ref[0, pl.ds(rr, 1), :], 0.0))
        q1 = jnp.sum(jnp.where(lane_m, rot_ref[1, pl.ds(rr, 1), :], 0.0))
        q2 = jnp.sum(jnp.where(lane_m, rot_ref[2, pl.ds(rr, 1), :], 0.0))
        u0 = jnp.sum(jnp.where(lane_m, trans_ref[0, pl.ds(rr, 1), :], 0.0))
        u1 = jnp.sum(jnp.where(lane_m, trans_ref[1, pl.ds(rr, 1), :], 0.0))
        u2 = jnp.sum(jnp.where(lane_m, trans_ref[2, pl.ds(rr, 1), :], 0.0))
        bvec = jnp.where(li == 0, b0,
               jnp.where(li == 1, b1,
               jnp.where(li == 2, b2,
               jnp.where(li == 3, b3, 0.0))))
        b_out[pl.ds(t, 1), :] = jnp.where(valid, bvec, -1.0)
        s_out[pl.ds(t, 1), :] = jnp.where(li == 0,
                                          jnp.where(valid, m, -1.0), 0.0)
        l_out[pl.ds(t, 1), :] = jnp.where(li == 0,
                                          jnp.where(valid, cls,
                                                    jnp.int32(-1)), 0)
        rvec = jnp.where(li == 0, q0,
               jnp.where(li == 1, q1,
               jnp.where(li == 2, q2, 0.0)))
        r_out[pl.ds(t, 1), :] = jnp.where(valid, rvec, -1.0)
        tvec = jnp.where(li == 0, u0,
               jnp.where(li == 1, u1,
               jnp.where(li == 2, u2, 0.0)))
        t_out[pl.ds(t, 1), :] = jnp.where(valid, tvec, -1.0)
        return carry

    jax.lax.fori_loop(0, _MAX_DET, pick, 0)


def _pad_cols(a_t, nrows):
    d = a_t.shape[0]
    out = jnp.zeros((d, _NPAD), a_t.dtype).at[:, :_N].set(a_t)
    return out.reshape(d, nrows, 128)


def kernel(boxes, classification, rotation, translation):
    clsP = jnp.full((_NUM_CLASSES, _NPAD), _NEG, jnp.float32)
    clsP = clsP.at[:, :_N].set(classification.T)
    clsP = clsP.reshape(_NUM_CLASSES, _ROWS, 128)
    boxesP = _pad_cols(boxes.T, _ROWS)
    rotP = _pad_cols(rotation.T, _ROWS)
    transP = _pad_cols(translation.T, _ROWS)

    scores_all, idx_all = pl.pallas_call(
        _phase1_body,
        grid=(_NUM_CLASSES,),
        in_specs=[
            pl.BlockSpec((1, _ROWS, 128), lambda c: (c, 0, 0)),
            pl.BlockSpec((4, _ROWS, 128), lambda c: (0, 0, 0)),
        ],
        out_specs=[
            pl.BlockSpec((1, 1, _KPAD), lambda c: (c, 0, 0)),
            pl.BlockSpec((1, 1, _KPAD), lambda c: (c, 0, 0)),
        ],
        out_shape=[
            jax.ShapeDtypeStruct((_NUM_CLASSES, 1, _KPAD), jnp.float32),
            jax.ShapeDtypeStruct((_NUM_CLASSES, 1, _KPAD), jnp.int32),
        ],
        scratch_shapes=[
            pltpu.VMEM((192, 128), jnp.float32),
            pltpu.VMEM((_KPAD, 128), jnp.float32),
            pltpu.VMEM((_KPAD, _KPAD), jnp.float32),
        ],
    )(clsP, boxesP)

    sc = scores_all.reshape(_NUM_CLASSES * _KPAD // 128, 128)
    ix = idx_all.reshape(_NUM_CLASSES * _KPAD // 128, 128).astype(jnp.float32)

    b, s, l, r, t = pl.pallas_call(
        _phase2_body,
        out_shape=[
            jax.ShapeDtypeStruct((104, 128), jnp.float32),
            jax.ShapeDtypeStruct((104, 128), jnp.float32),
            jax.ShapeDtypeStruct((104, 128), jnp.int32),
            jax.ShapeDtypeStruct((104, 128), jnp.float32),
            jax.ShapeDtypeStruct((104, 128), jnp.float32),
        ],
        scratch_shapes=[pltpu.VMEM((32, 128), jnp.float32)],
    )(sc, ix, boxesP, rotP, transP)

    boxes_out = b[:_MAX_DET, :4]
    scores_out = s[:_MAX_DET, 0]
    labels_out = l[:_MAX_DET, 0].astype(jnp.int64)
    rotation_out = r[:_MAX_DET, :3]
    translation_out = t[:_MAX_DET, :3]
    return (boxes_out, scores_out, labels_out, rotation_out, translation_out)


# speedup vs baseline: 1.3024x; 1.3024x over previous
import functools

import jax
import jax.numpy as jnp
from jax.experimental import pallas as pl
from jax.experimental.pallas import tpu as pltpu

_N = 20000
_NPAD = 20480
_ROWS = 160
_NUM_CLASSES = 8
_K = 500
_KPAD = 512
_SCORE_THRESHOLD = 0.01
_NMS_THRESHOLD = 0.5
_MAX_DET = 100
_NEG = -3.0e38
_BIGI = 2**30


def _phase1_body(cls_ref, boxes_ref, scores_out_ref, idx_out_ref,
                 x_scr, s_scr, iou_scr):
    x0 = jnp.concatenate(
        [cls_ref[0], jnp.full((32, 128), _NEG, jnp.float32)], axis=0)
    x_scr[...] = x0
    s_scr[...] = jnp.zeros((_KPAD, 128), jnp.float32)

    li = jax.lax.broadcasted_iota(jnp.int32, (1, 128), 1)
    ri24 = jax.lax.broadcasted_iota(jnp.int32, (24, 128), 0)
    ri8 = jax.lax.broadcasted_iota(jnp.int32, (8, 128), 0)
    li8 = jax.lax.broadcasted_iota(jnp.int32, (8, 128), 1)

    flatiota = ri24
    flat192 = jax.lax.broadcasted_iota(jnp.int32, (192, 128), 0) * 128 + \
        jax.lax.broadcasted_iota(jnp.int32, (192, 128), 1)

    def extract(t, carry):
        x = x_scr[...]
        m = jnp.max(x)
        f = jnp.min(jnp.where(x == m, flat192, _BIGI))
        r = f >> 7
        l = f & 127
        xrow = x_scr[pl.ds(r, 1), :]
        x_scr[pl.ds(r, 1), :] = jnp.where(li == l, _NEG, xrow)
        lane_m = (li == l)
        c0 = jnp.sum(jnp.where(lane_m, boxes_ref[0, pl.ds(r, 1), :], 0.0))
        c1 = jnp.sum(jnp.where(lane_m, boxes_ref[1, pl.ds(r, 1), :], 0.0))
        c2 = jnp.sum(jnp.where(lane_m, boxes_ref[2, pl.ds(r, 1), :], 0.0))
        c3 = jnp.sum(jnp.where(lane_m, boxes_ref[3, pl.ds(r, 1), :], 0.0))
        svec = jnp.where(li == 0, c0,
               jnp.where(li == 1, c1,
               jnp.where(li == 2, c2,
               jnp.where(li == 3, c3,
               jnp.where(li == 4, m,
               jnp.where(li == 5, f.astype(jnp.float32), 0.0))))))
        s_scr[pl.ds(t, 1), :] = svec
        return carry

    jax.lax.fori_loop(0, _K, extract, 0)

    s = s_scr[...]
    st = s.T
    x1r = st[0:1, :]
    y1r = st[1:2, :]
    x2r = st[2:3, :]
    y2r = st[3:4, :]
    vals = st[4:5, :]
    idxr = st[5:6, :]
    x1c = s[:, 0:1]
    y1c = s[:, 1:2]
    x2c = s[:, 2:3]
    y2c = s[:, 3:4]

    area_r = jnp.maximum(x2r - x1r, 0.0) * jnp.maximum(y2r - y1r, 0.0)
    area_c = jnp.maximum(x2c - x1c, 0.0) * jnp.maximum(y2c - y1c, 0.0)
    sh = (_KPAD, _KPAD)
    xx1 = jnp.maximum(jnp.broadcast_to(x1c, sh), jnp.broadcast_to(x1r, sh))
    yy1 = jnp.maximum(jnp.broadcast_to(y1c, sh), jnp.broadcast_to(y1r, sh))
    xx2 = jnp.minimum(jnp.broadcast_to(x2c, sh), jnp.broadcast_to(x2r, sh))
    yy2 = jnp.minimum(jnp.broadcast_to(y2c, sh), jnp.broadcast_to(y2r, sh))
    inter = jnp.maximum(xx2 - xx1, 0.0) * jnp.maximum(yy2 - yy1, 0.0)
    union = jnp.broadcast_to(area_c, sh) + jnp.broadcast_to(area_r, sh) - inter
    iou_scr[...] = inter / jnp.maximum(union, 1e-8)

    li512 = jax.lax.broadcasted_iota(jnp.int32, (1, _KPAD), 1)
    keep0 = jnp.where(vals > _SCORE_THRESHOLD, 1.0, 0.0)

    def nms_step(i, keep):
        row = iou_scr[pl.ds(i, 1), :]
        keep_i = jnp.max(jnp.where(li512 == i, keep, 0.0))
        suppress = (row > _NMS_THRESHOLD) & (li512 > i) & (keep_i > 0.5)
        return jnp.where(suppress, 0.0, keep)

    keep = jax.lax.fori_loop(0, _K, nms_step, keep0)

    scores_out_ref[0] = jnp.where(keep > 0.5, vals, -1.0)
    idx_out_ref[0] = idxr.astype(jnp.int32)


def _phase2_body(sc_ref, ix_ref, boxes_ref, rot_ref, trans_ref,
                 b_out, s_out, l_out, r_out, t_out, y_scr):
    y_scr[...] = sc_ref[...]
    li = jax.lax.broadcasted_iota(jnp.int32, (1, 128), 1)
    flatiota = jax.lax.broadcasted_iota(jnp.int32, (32, 128), 0) * 128 + \
        jax.lax.broadcasted_iota(jnp.int32, (32, 128), 1)

    def pick(t, carry):
        y = y_scr[...]
        m = jnp.max(y)
        p = jnp.min(jnp.where(y == m, flatiota, _BIGI))
        pr = p >> 7
        pln = p & 127
        yrow = y_scr[pl.ds(pr, 1), :]
        y_scr[pl.ds(pr, 1), :] = jnp.where(li == pln, _NEG, yrow)
        cls = p >> 9
        f = jnp.sum(jnp.where(li == pln, ix_ref[pl.ds(pr, 1), :], 0.0)
                    ).astype(jnp.int32)
        valid = m > -0.5
        rr = f >> 7
        rl = f & 127
        lane_m = (li == rl)
        b0 = jnp.sum(jnp.where(lane_m, boxes_ref[0, pl.ds(rr, 1), :], 0.0))
        b1 = jnp.sum(jnp.where(lane_m, boxes_ref[1, pl.ds(rr, 1), :], 0.0))
        b2 = jnp.sum(jnp.where(lane_m, boxes_ref[2, pl.ds(rr, 1), :], 0.0))
        b3 = jnp.sum(jnp.where(lane_m, boxes_ref[3, pl.ds(rr, 1), :], 0.0))
        q0 = jnp.sum(jnp.where(lane_m, rot_ref[0, pl.ds(rr, 1), :], 0.0))
        q1 = jnp.sum(jnp.where(lane_m, rot_ref[1, pl.ds(rr, 1), :], 0.0))
        q2 = jnp.sum(jnp.where(lane_m, rot_ref[2, pl.ds(rr, 1), :], 0.0))
        u0 = jnp.sum(jnp.where(lane_m, trans_ref[0, pl.ds(rr, 1), :], 0.0))
        u1 = jnp.sum(jnp.where(lane_m, trans_ref[1, pl.ds(rr, 1), :], 0.0))
        u2 = jnp.sum(jnp.where(lane_m, trans_ref[2, pl.ds(rr, 1), :], 0.0))
        bvec = jnp.where(li == 0, b0,
               jnp.where(li == 1, b1,
               jnp.where(li == 2, b2,
               jnp.where(li == 3, b3, 0.0))))
        b_out[pl.ds(t, 1), :] = jnp.where(valid, bvec, -1.0)
        s_out[pl.ds(t, 1), :] = jnp.where(li == 0,
                                          jnp.where(valid, m, -1.0), 0.0)
        l_out[pl.ds(t, 1), :] = jnp.where(li == 0,
                                          jnp.where(valid, cls,
                                                    jnp.int32(-1)), 0)
        rvec = jnp.where(li == 0, q0,
               jnp.where(li == 1, q1,
               jnp.where(li == 2, q2, 0.0)))
        r_out[pl.ds(t, 1), :] = jnp.where(valid, rvec, -1.0)
        tvec = jnp.where(li == 0, u0,
               jnp.where(li == 1, u1,
               jnp.where(li == 2, u2, 0.0)))
        t_out[pl.ds(t, 1), :] = jnp.where(valid, tvec, -1.0)
        return carry

    jax.lax.fori_loop(0, _MAX_DET, pick, 0)


def _pad_cols(a_t, nrows):
    d = a_t.shape[0]
    out = jnp.zeros((d, _NPAD), a_t.dtype).at[:, :_N].set(a_t)
    return out.reshape(d, nrows, 128)


def kernel(boxes, classification, rotation, translation):
    clsP = jnp.full((_NUM_CLASSES, _NPAD), _NEG, jnp.float32)
    clsP = clsP.at[:, :_N].set(classification.T)
    clsP = clsP.reshape(_NUM_CLASSES, _ROWS, 128)
    boxesP = _pad_cols(boxes.T, _ROWS)
    rotP = _pad_cols(rotation.T, _ROWS)
    transP = _pad_cols(translation.T, _ROWS)

    scores_all, idx_all = pl.pallas_call(
        _phase1_body,
        grid=(_NUM_CLASSES,),
        in_specs=[
            pl.BlockSpec((1, _ROWS, 128), lambda c: (c, 0, 0)),
            pl.BlockSpec((4, _ROWS, 128), lambda c: (0, 0, 0)),
        ],
        out_specs=[
            pl.BlockSpec((1, 1, _KPAD), lambda c: (c, 0, 0)),
            pl.BlockSpec((1, 1, _KPAD), lambda c: (c, 0, 0)),
        ],
        out_shape=[
            jax.ShapeDtypeStruct((_NUM_CLASSES, 1, _KPAD), jnp.float32),
            jax.ShapeDtypeStruct((_NUM_CLASSES, 1, _KPAD), jnp.int32),
        ],
        scratch_shapes=[
            pltpu.VMEM((192, 128), jnp.float32),
            pltpu.VMEM((_KPAD, 128), jnp.float32),
            pltpu.VMEM((_KPAD, _KPAD), jnp.float32),
        ],
    )(clsP, boxesP)

    sc = scores_all.reshape(_NUM_CLASSES * _KPAD // 128, 128)
    ix = idx_all.reshape(_NUM_CLASSES * _KPAD // 128, 128).astype(jnp.float32)

    b, s, l, r, t = pl.pallas_call(
        _phase2_body,
        out_shape=[
            jax.ShapeDtypeStruct((104, 128), jnp.float32),
            jax.ShapeDtypeStruct((104, 128), jnp.float32),
            jax.ShapeDtypeStruct((104, 128), jnp.int32),
            jax.ShapeDtypeStruct((104, 128), jnp.float32),
            jax.ShapeDtypeStruct((104, 128), jnp.float32),
        ],
        scratch_shapes=[pltpu.VMEM((32, 128), jnp.float32)],
    )(sc, ix, boxesP, rotP, transP)

    boxes_out = b[:_MAX_DET, :4]
    scores_out = s[:_MAX_DET, 0]
    labels_out = l[:_MAX_DET, 0].astype(jnp.int64)
    rotation_out = r[:_MAX_DET, :3]
    translation_out = t[:_MAX_DET, :3]
    return (boxes_out, scores_out, labels_out, rotation_out, translation_out)


# speedup vs baseline: 1.4371x; 1.1034x over previous
import functools

import jax
import jax.numpy as jnp
from jax.experimental import pallas as pl
from jax.experimental.pallas import tpu as pltpu

_N = 20000
_NPAD = 20480
_ROWS = 160
_NUM_CLASSES = 8
_K = 500
_KPAD = 512
_SCORE_THRESHOLD = 0.01
_NMS_THRESHOLD = 0.5
_MAX_DET = 100
_NEG = -3.0e38
_BIGI = 2**30


def _phase1_body(cls_ref, boxes_ref, scores_out_ref, idx_out_ref,
                 s_scr, iou_scr):
    x0 = jnp.concatenate(
        [cls_ref[0], jnp.full((32, 128), _NEG, jnp.float32)], axis=0)
    s_scr[...] = jnp.zeros((_KPAD, 128), jnp.float32)

    li = jax.lax.broadcasted_iota(jnp.int32, (1, 128), 1)
    flat192 = jax.lax.broadcasted_iota(jnp.int32, (192, 128), 0) * 128 + \
        jax.lax.broadcasted_iota(jnp.int32, (192, 128), 1)

    def pick_one(x):
        m = jnp.max(x)
        f = jnp.min(jnp.where(x == m, flat192, _BIGI))
        r = f >> 7
        l = f & 127
        lane_m = (li == l)
        c0 = jnp.sum(jnp.where(lane_m, boxes_ref[0, pl.ds(r, 1), :], 0.0))
        c1 = jnp.sum(jnp.where(lane_m, boxes_ref[1, pl.ds(r, 1), :], 0.0))
        c2 = jnp.sum(jnp.where(lane_m, boxes_ref[2, pl.ds(r, 1), :], 0.0))
        c3 = jnp.sum(jnp.where(lane_m, boxes_ref[3, pl.ds(r, 1), :], 0.0))
        svec = jnp.where(li == 0, c0,
               jnp.where(li == 1, c1,
               jnp.where(li == 2, c2,
               jnp.where(li == 3, c3,
               jnp.where(li == 4, m,
               jnp.where(li == 5, f.astype(jnp.float32), 0.0))))))
        return jnp.where(flat192 == f, _NEG, x), svec

    def extract(t, x):
        x, svec_a = pick_one(x)
        x, svec_b = pick_one(x)
        s_scr[pl.ds(2 * t, 1), :] = svec_a
        s_scr[pl.ds(2 * t + 1, 1), :] = svec_b
        return x

    jax.lax.fori_loop(0, _K // 2, extract, x0)

    s = s_scr[...]
    st = s.T
    x1r = st[0:1, :]
    y1r = st[1:2, :]
    x2r = st[2:3, :]
    y2r = st[3:4, :]
    vals = st[4:5, :]
    idxr = st[5:6, :]
    x1c = s[:, 0:1]
    y1c = s[:, 1:2]
    x2c = s[:, 2:3]
    y2c = s[:, 3:4]

    area_r = jnp.maximum(x2r - x1r, 0.0) * jnp.maximum(y2r - y1r, 0.0)
    area_c = jnp.maximum(x2c - x1c, 0.0) * jnp.maximum(y2c - y1c, 0.0)
    sh = (_KPAD, _KPAD)
    xx1 = jnp.maximum(jnp.broadcast_to(x1c, sh), jnp.broadcast_to(x1r, sh))
    yy1 = jnp.maximum(jnp.broadcast_to(y1c, sh), jnp.broadcast_to(y1r, sh))
    xx2 = jnp.minimum(jnp.broadcast_to(x2c, sh), jnp.broadcast_to(x2r, sh))
    yy2 = jnp.minimum(jnp.broadcast_to(y2c, sh), jnp.broadcast_to(y2r, sh))
    inter = jnp.maximum(xx2 - xx1, 0.0) * jnp.maximum(yy2 - yy1, 0.0)
    union = jnp.broadcast_to(area_c, sh) + jnp.broadcast_to(area_r, sh) - inter
    iou_scr[...] = inter / jnp.maximum(union, 1e-8)

    li512 = jax.lax.broadcasted_iota(jnp.int32, (1, _KPAD), 1)
    keep0 = jnp.where(vals > _SCORE_THRESHOLD, 1.0, 0.0)

    def nms_step(i, keep):
        row = iou_scr[pl.ds(i, 1), :]
        keep_i = jnp.max(jnp.where(li512 == i, keep, 0.0))
        suppress = (row > _NMS_THRESHOLD) & (li512 > i) & (keep_i > 0.5)
        return jnp.where(suppress, 0.0, keep)

    keep = jax.lax.fori_loop(0, _K, nms_step, keep0)

    scores_out_ref[0] = jnp.where(keep > 0.5, vals, -1.0)
    idx_out_ref[0] = idxr.astype(jnp.int32)


def _phase2_body(sc_ref, ix_ref, boxes_ref, rot_ref, trans_ref,
                 b_out, s_out, l_out, r_out, t_out, y_scr):
    y_scr[...] = sc_ref[...]
    li = jax.lax.broadcasted_iota(jnp.int32, (1, 128), 1)
    flatiota = jax.lax.broadcasted_iota(jnp.int32, (32, 128), 0) * 128 + \
        jax.lax.broadcasted_iota(jnp.int32, (32, 128), 1)

    def pick(t, carry):
        y = y_scr[...]
        m = jnp.max(y)
        p = jnp.min(jnp.where(y == m, flatiota, _BIGI))
        pr = p >> 7
        pln = p & 127
        yrow = y_scr[pl.ds(pr, 1), :]
        y_scr[pl.ds(pr, 1), :] = jnp.where(li == pln, _NEG, yrow)
        cls = p >> 9
        f = jnp.sum(jnp.where(li == pln, ix_ref[pl.ds(pr, 1), :], 0.0)
                    ).astype(jnp.int32)
        valid = m > -0.5
        rr = f >> 7
        rl = f & 127
        lane_m = (li == rl)
        b0 = jnp.sum(jnp.where(lane_m, boxes_ref[0, pl.ds(rr, 1), :], 0.0))
        b1 = jnp.sum(jnp.where(lane_m, boxes_ref[1, pl.ds(rr, 1), :], 0.0))
        b2 = jnp.sum(jnp.where(lane_m, boxes_ref[2, pl.ds(rr, 1), :], 0.0))
        b3 = jnp.sum(jnp.where(lane_m, boxes_ref[3, pl.ds(rr, 1), :], 0.0))
        q0 = jnp.sum(jnp.where(lane_m, rot_ref[0, pl.ds(rr, 1), :], 0.0))
        q1 = jnp.sum(jnp.where(lane_m, rot_ref[1, pl.ds(rr, 1), :], 0.0))
        q2 = jnp.sum(jnp.where(lane_m, rot_ref[2, pl.ds(rr, 1), :], 0.0))
        u0 = jnp.sum(jnp.where(lane_m, trans_ref[0, pl.ds(rr, 1), :], 0.0))
        u1 = jnp.sum(jnp.where(lane_m, trans_ref[1, pl.ds(rr, 1), :], 0.0))
        u2 = jnp.sum(jnp.where(lane_m, trans_ref[2, pl.ds(rr, 1), :], 0.0))
        bvec = jnp.where(li == 0, b0,
               jnp.where(li == 1, b1,
               jnp.where(li == 2, b2,
               jnp.where(li == 3, b3, 0.0))))
        b_out[pl.ds(t, 1), :] = jnp.where(valid, bvec, -1.0)
        s_out[pl.ds(t, 1), :] = jnp.where(li == 0,
                                          jnp.where(valid, m, -1.0), 0.0)
        l_out[pl.ds(t, 1), :] = jnp.where(li == 0,
                                          jnp.where(valid, cls,
                                                    jnp.int32(-1)), 0)
        rvec = jnp.where(li == 0, q0,
               jnp.where(li == 1, q1,
               jnp.where(li == 2, q2, 0.0)))
        r_out[pl.ds(t, 1), :] = jnp.where(valid, rvec, -1.0)
        tvec = jnp.where(li == 0, u0,
               jnp.where(li == 1, u1,
               jnp.where(li == 2, u2, 0.0)))
        t_out[pl.ds(t, 1), :] = jnp.where(valid, tvec, -1.0)
        return carry

    jax.lax.fori_loop(0, _MAX_DET, pick, 0)


def _pad_cols(a_t, nrows):
    d = a_t.shape[0]
    out = jnp.zeros((d, _NPAD), a_t.dtype).at[:, :_N].set(a_t)
    return out.reshape(d, nrows, 128)


def kernel(boxes, classification, rotation, translation):
    clsP = jnp.full((_NUM_CLASSES, _NPAD), _NEG, jnp.float32)
    clsP = clsP.at[:, :_N].set(classification.T)
    clsP = clsP.reshape(_NUM_CLASSES, _ROWS, 128)
    boxesP = _pad_cols(boxes.T, _ROWS)
    rotP = _pad_cols(rotation.T, _ROWS)
    transP = _pad_cols(translation.T, _ROWS)

    scores_all, idx_all = pl.pallas_call(
        _phase1_body,
        grid=(_NUM_CLASSES,),
        in_specs=[
            pl.BlockSpec((1, _ROWS, 128), lambda c: (c, 0, 0)),
            pl.BlockSpec((4, _ROWS, 128), lambda c: (0, 0, 0)),
        ],
        out_specs=[
            pl.BlockSpec((1, 1, _KPAD), lambda c: (c, 0, 0)),
            pl.BlockSpec((1, 1, _KPAD), lambda c: (c, 0, 0)),
        ],
        out_shape=[
            jax.ShapeDtypeStruct((_NUM_CLASSES, 1, _KPAD), jnp.float32),
            jax.ShapeDtypeStruct((_NUM_CLASSES, 1, _KPAD), jnp.int32),
        ],
        scratch_shapes=[
            pltpu.VMEM((_KPAD, 128), jnp.float32),
            pltpu.VMEM((_KPAD, _KPAD), jnp.float32),
        ],
    )(clsP, boxesP)

    sc = scores_all.reshape(_NUM_CLASSES * _KPAD // 128, 128)
    ix = idx_all.reshape(_NUM_CLASSES * _KPAD // 128, 128).astype(jnp.float32)

    b, s, l, r, t = pl.pallas_call(
        _phase2_body,
        out_shape=[
            jax.ShapeDtypeStruct((104, 128), jnp.float32),
            jax.ShapeDtypeStruct((104, 128), jnp.float32),
            jax.ShapeDtypeStruct((104, 128), jnp.int32),
            jax.ShapeDtypeStruct((104, 128), jnp.float32),
            jax.ShapeDtypeStruct((104, 128), jnp.float32),
        ],
        scratch_shapes=[pltpu.VMEM((32, 128), jnp.float32)],
    )(sc, ix, boxesP, rotP, transP)

    boxes_out = b[:_MAX_DET, :4]
    scores_out = s[:_MAX_DET, 0]
    labels_out = l[:_MAX_DET, 0].astype(jnp.int64)
    rotation_out = r[:_MAX_DET, :3]
    translation_out = t[:_MAX_DET, :3]
    return (boxes_out, scores_out, labels_out, rotation_out, translation_out)


# speedup vs baseline: 1.6117x; 1.1215x over previous
import functools

import jax
import jax.numpy as jnp
from jax.experimental import pallas as pl
from jax.experimental.pallas import tpu as pltpu

_N = 20000
_NPAD = 20480
_ROWS = 160
_NUM_CLASSES = 8
_K = 500
_KPAD = 512
_SCORE_THRESHOLD = 0.01
_NMS_THRESHOLD = 0.5
_MAX_DET = 100
_NEG = -3.0e38
_BIGI = 2**30


def _phase1_body(cls_ref, boxes_ref, scores_out_ref, idx_out_ref,
                 s_scr, iou_scr):
    x0 = jnp.concatenate(
        [cls_ref[0], jnp.full((32, 128), _NEG, jnp.float32)], axis=0)
    s_scr[...] = jnp.zeros((_KPAD, 128), jnp.float32)

    li = jax.lax.broadcasted_iota(jnp.int32, (1, 128), 1)
    flat192 = jax.lax.broadcasted_iota(jnp.int32, (192, 128), 0) * 128 + \
        jax.lax.broadcasted_iota(jnp.int32, (192, 128), 1)

    def pick_one(x):
        m = jnp.max(x)
        f = jnp.min(jnp.where(x == m, flat192, _BIGI))
        r = f >> 7
        l = f & 127
        lane_m = (li == l)
        c0 = jnp.sum(jnp.where(lane_m, boxes_ref[0, pl.ds(r, 1), :], 0.0))
        c1 = jnp.sum(jnp.where(lane_m, boxes_ref[1, pl.ds(r, 1), :], 0.0))
        c2 = jnp.sum(jnp.where(lane_m, boxes_ref[2, pl.ds(r, 1), :], 0.0))
        c3 = jnp.sum(jnp.where(lane_m, boxes_ref[3, pl.ds(r, 1), :], 0.0))
        svec = jnp.where(li == 0, c0,
               jnp.where(li == 1, c1,
               jnp.where(li == 2, c2,
               jnp.where(li == 3, c3,
               jnp.where(li == 4, m,
               jnp.where(li == 5, f.astype(jnp.float32), 0.0))))))
        return jnp.where(flat192 == f, _NEG, x), svec

    def extract(t, x):
        x, svec_a = pick_one(x)
        x, svec_b = pick_one(x)
        s_scr[pl.ds(2 * t, 1), :] = svec_a
        s_scr[pl.ds(2 * t + 1, 1), :] = svec_b
        return x

    jax.lax.fori_loop(0, _K // 2, extract, x0)

    s = s_scr[...]
    st = s.T
    x1r = st[0:1, :]
    y1r = st[1:2, :]
    x2r = st[2:3, :]
    y2r = st[3:4, :]
    vals = st[4:5, :]
    idxr = st[5:6, :]
    x1c = s[:, 0:1]
    y1c = s[:, 1:2]
    x2c = s[:, 2:3]
    y2c = s[:, 3:4]

    area_r = jnp.maximum(x2r - x1r, 0.0) * jnp.maximum(y2r - y1r, 0.0)
    area_c = jnp.maximum(x2c - x1c, 0.0) * jnp.maximum(y2c - y1c, 0.0)
    sh = (_KPAD, _KPAD)
    xx1 = jnp.maximum(jnp.broadcast_to(x1c, sh), jnp.broadcast_to(x1r, sh))
    yy1 = jnp.maximum(jnp.broadcast_to(y1c, sh), jnp.broadcast_to(y1r, sh))
    xx2 = jnp.minimum(jnp.broadcast_to(x2c, sh), jnp.broadcast_to(x2r, sh))
    yy2 = jnp.minimum(jnp.broadcast_to(y2c, sh), jnp.broadcast_to(y2r, sh))
    inter = jnp.maximum(xx2 - xx1, 0.0) * jnp.maximum(yy2 - yy1, 0.0)
    union = jnp.broadcast_to(area_c, sh) + jnp.broadcast_to(area_r, sh) - inter
    iou_scr[...] = inter / jnp.maximum(union, 1e-8)

    li512 = jax.lax.broadcasted_iota(jnp.int32, (1, _KPAD), 1)
    keep0 = jnp.where(vals > _SCORE_THRESHOLD, 1.0, 0.0)

    def nms_step(t, keep):
        i = 2 * t
        row_a = iou_scr[pl.ds(i, 1), :]
        row_b = iou_scr[pl.ds(i + 1, 1), :]
        e_a = jnp.max(jnp.where(li512 == i, keep, 0.0)) > 0.5
        kb_pre = jnp.max(jnp.where(li512 == i + 1, keep, 0.0)) > 0.5
        iou_ab = jnp.max(jnp.where(li512 == i + 1, row_a, 0.0))
        e_b = kb_pre & jnp.logical_not(e_a & (iou_ab > _NMS_THRESHOLD))
        sup_a = (row_a > _NMS_THRESHOLD) & (li512 > i) & e_a
        sup_b = (row_b > _NMS_THRESHOLD) & (li512 > i + 1) & e_b
        return jnp.where(sup_a | sup_b, 0.0, keep)

    keep = jax.lax.fori_loop(0, _K // 2, nms_step, keep0)

    scores_out_ref[0] = jnp.where(keep > 0.5, vals, -1.0)
    idx_out_ref[0] = idxr.astype(jnp.int32)


def _phase2_body(sc_ref, ix_ref, boxes_ref, rot_ref, trans_ref,
                 b_out, s_out, l_out, r_out, t_out, y_scr):
    y_scr[...] = sc_ref[...]
    li = jax.lax.broadcasted_iota(jnp.int32, (1, 128), 1)
    flatiota = jax.lax.broadcasted_iota(jnp.int32, (32, 128), 0) * 128 + \
        jax.lax.broadcasted_iota(jnp.int32, (32, 128), 1)

    def pick(t, carry):
        y = y_scr[...]
        m = jnp.max(y)
        p = jnp.min(jnp.where(y == m, flatiota, _BIGI))
        pr = p >> 7
        pln = p & 127
        yrow = y_scr[pl.ds(pr, 1), :]
        y_scr[pl.ds(pr, 1), :] = jnp.where(li == pln, _NEG, yrow)
        cls = p >> 9
        f = jnp.sum(jnp.where(li == pln, ix_ref[pl.ds(pr, 1), :], 0.0)
                    ).astype(jnp.int32)
        valid = m > -0.5
        rr = f >> 7
        rl = f & 127
        lane_m = (li == rl)
        b0 = jnp.sum(jnp.where(lane_m, boxes_ref[0, pl.ds(rr, 1), :], 0.0))
        b1 = jnp.sum(jnp.where(lane_m, boxes_ref[1, pl.ds(rr, 1), :], 0.0))
        b2 = jnp.sum(jnp.where(lane_m, boxes_ref[2, pl.ds(rr, 1), :], 0.0))
        b3 = jnp.sum(jnp.where(lane_m, boxes_ref[3, pl.ds(rr, 1), :], 0.0))
        q0 = jnp.sum(jnp.where(lane_m, rot_ref[0, pl.ds(rr, 1), :], 0.0))
        q1 = jnp.sum(jnp.where(lane_m, rot_ref[1, pl.ds(rr, 1), :], 0.0))
        q2 = jnp.sum(jnp.where(lane_m, rot_ref[2, pl.ds(rr, 1), :], 0.0))
        u0 = jnp.sum(jnp.where(lane_m, trans_ref[0, pl.ds(rr, 1), :], 0.0))
        u1 = jnp.sum(jnp.where(lane_m, trans_ref[1, pl.ds(rr, 1), :], 0.0))
        u2 = jnp.sum(jnp.where(lane_m, trans_ref[2, pl.ds(rr, 1), :], 0.0))
        bvec = jnp.where(li == 0, b0,
               jnp.where(li == 1, b1,
               jnp.where(li == 2, b2,
               jnp.where(li == 3, b3, 0.0))))
        b_out[pl.ds(t, 1), :] = jnp.where(valid, bvec, -1.0)
        s_out[pl.ds(t, 1), :] = jnp.where(li == 0,
                                          jnp.where(valid, m, -1.0), 0.0)
        l_out[pl.ds(t, 1), :] = jnp.where(li == 0,
                                          jnp.where(valid, cls,
                                                    jnp.int32(-1)), 0)
        rvec = jnp.where(li == 0, q0,
               jnp.where(li == 1, q1,
               jnp.where(li == 2, q2, 0.0)))
        r_out[pl.ds(t, 1), :] = jnp.where(valid, rvec, -1.0)
        tvec = jnp.where(li == 0, u0,
               jnp.where(li == 1, u1,
               jnp.where(li == 2, u2, 0.0)))
        t_out[pl.ds(t, 1), :] = jnp.where(valid, tvec, -1.0)
        return carry

    jax.lax.fori_loop(0, _MAX_DET, pick, 0)


def _pad_cols(a_t, nrows):
    d = a_t.shape[0]
    out = jnp.zeros((d, _NPAD), a_t.dtype).at[:, :_N].set(a_t)
    return out.reshape(d, nrows, 128)


def kernel(boxes, classification, rotation, translation):
    clsP = jnp.full((_NUM_CLASSES, _NPAD), _NEG, jnp.float32)
    clsP = clsP.at[:, :_N].set(classification.T)
    clsP = clsP.reshape(_NUM_CLASSES, _ROWS, 128)
    boxesP = _pad_cols(boxes.T, _ROWS)
    rotP = _pad_cols(rotation.T, _ROWS)
    transP = _pad_cols(translation.T, _ROWS)

    scores_all, idx_all = pl.pallas_call(
        _phase1_body,
        grid=(_NUM_CLASSES,),
        in_specs=[
            pl.BlockSpec((1, _ROWS, 128), lambda c: (c, 0, 0)),
            pl.BlockSpec((4, _ROWS, 128), lambda c: (0, 0, 0)),
        ],
        out_specs=[
            pl.BlockSpec((1, 1, _KPAD), lambda c: (c, 0, 0)),
            pl.BlockSpec((1, 1, _KPAD), lambda c: (c, 0, 0)),
        ],
        out_shape=[
            jax.ShapeDtypeStruct((_NUM_CLASSES, 1, _KPAD), jnp.float32),
            jax.ShapeDtypeStruct((_NUM_CLASSES, 1, _KPAD), jnp.int32),
        ],
        scratch_shapes=[
            pltpu.VMEM((_KPAD, 128), jnp.float32),
            pltpu.VMEM((_KPAD, _KPAD), jnp.float32),
        ],
    )(clsP, boxesP)

    sc = scores_all.reshape(_NUM_CLASSES * _KPAD // 128, 128)
    ix = idx_all.reshape(_NUM_CLASSES * _KPAD // 128, 128).astype(jnp.float32)

    b, s, l, r, t = pl.pallas_call(
        _phase2_body,
        out_shape=[
            jax.ShapeDtypeStruct((104, 128), jnp.float32),
            jax.ShapeDtypeStruct((104, 128), jnp.float32),
            jax.ShapeDtypeStruct((104, 128), jnp.int32),
            jax.ShapeDtypeStruct((104, 128), jnp.float32),
            jax.ShapeDtypeStruct((104, 128), jnp.float32),
        ],
        scratch_shapes=[pltpu.VMEM((32, 128), jnp.float32)],
    )(sc, ix, boxesP, rotP, transP)

    boxes_out = b[:_MAX_DET, :4]
    scores_out = s[:_MAX_DET, 0]
    labels_out = l[:_MAX_DET, 0].astype(jnp.int64)
    rotation_out = r[:_MAX_DET, :3]
    translation_out = t[:_MAX_DET, :3]
    return (boxes_out, scores_out, labels_out, rotation_out, translation_out)


# speedup vs baseline: 1.8582x; 1.1530x over previous
import functools

import jax
import jax.numpy as jnp
from jax.experimental import pallas as pl
from jax.experimental.pallas import tpu as pltpu

_N = 20000
_NPAD = 20480
_ROWS = 160
_NUM_CLASSES = 8
_K = 500
_KPAD = 512
_SCORE_THRESHOLD = 0.01
_NMS_THRESHOLD = 0.5
_MAX_DET = 100
_NEG = -3.0e38
_BIGI = 2**30


def _phase1_body(cls_ref, boxes_ref, scores_out_ref, idx_out_ref,
                 s_scr, iou_scr):
    pad = jnp.full((32, 128), _NEG, jnp.float32)
    xa0 = jnp.concatenate([cls_ref[0], pad], axis=0)
    xb0 = jnp.concatenate([cls_ref[1], pad], axis=0)
    s_scr[...] = jnp.zeros((2, _KPAD, 128), jnp.float32)

    li = jax.lax.broadcasted_iota(jnp.int32, (1, 128), 1)
    flat192 = jax.lax.broadcasted_iota(jnp.int32, (192, 128), 0) * 128 + \
        jax.lax.broadcasted_iota(jnp.int32, (192, 128), 1)

    def pick_one(x):
        m = jnp.max(x)
        f = jnp.min(jnp.where(x == m, flat192, _BIGI))
        r = f >> 7
        l = f & 127
        lane_m = (li == l)
        c0 = jnp.sum(jnp.where(lane_m, boxes_ref[0, pl.ds(r, 1), :], 0.0))
        c1 = jnp.sum(jnp.where(lane_m, boxes_ref[1, pl.ds(r, 1), :], 0.0))
        c2 = jnp.sum(jnp.where(lane_m, boxes_ref[2, pl.ds(r, 1), :], 0.0))
        c3 = jnp.sum(jnp.where(lane_m, boxes_ref[3, pl.ds(r, 1), :], 0.0))
        svec = jnp.where(li == 0, c0,
               jnp.where(li == 1, c1,
               jnp.where(li == 2, c2,
               jnp.where(li == 3, c3,
               jnp.where(li == 4, m,
               jnp.where(li == 5, f.astype(jnp.float32), 0.0))))))
        return jnp.where(flat192 == f, _NEG, x), svec

    def extract(t, xs):
        xa, xb = xs
        xa, sa1 = pick_one(xa)
        xb, sb1 = pick_one(xb)
        xa, sa2 = pick_one(xa)
        xb, sb2 = pick_one(xb)
        s_scr[0, pl.ds(2 * t, 1), :] = sa1
        s_scr[1, pl.ds(2 * t, 1), :] = sb1
        s_scr[0, pl.ds(2 * t + 1, 1), :] = sa2
        s_scr[1, pl.ds(2 * t + 1, 1), :] = sb2
        return (xa, xb)

    jax.lax.fori_loop(0, _K // 2, extract, (xa0, xb0))

    li512 = jax.lax.broadcasted_iota(jnp.int32, (1, _KPAD), 1)
    sh = (_KPAD, _KPAD)
    keeps0 = []
    valss = []
    idxrs = []
    for c in range(2):
        s = s_scr[c]
        st = s.T
        x1r = st[0:1, :]
        y1r = st[1:2, :]
        x2r = st[2:3, :]
        y2r = st[3:4, :]
        vals = st[4:5, :]
        idxrs.append(st[5:6, :])
        valss.append(vals)
        x1c = s[:, 0:1]
        y1c = s[:, 1:2]
        x2c = s[:, 2:3]
        y2c = s[:, 3:4]
        area_r = jnp.maximum(x2r - x1r, 0.0) * jnp.maximum(y2r - y1r, 0.0)
        area_c = jnp.maximum(x2c - x1c, 0.0) * jnp.maximum(y2c - y1c, 0.0)
        xx1 = jnp.maximum(jnp.broadcast_to(x1c, sh), jnp.broadcast_to(x1r, sh))
        yy1 = jnp.maximum(jnp.broadcast_to(y1c, sh), jnp.broadcast_to(y1r, sh))
        xx2 = jnp.minimum(jnp.broadcast_to(x2c, sh), jnp.broadcast_to(x2r, sh))
        yy2 = jnp.minimum(jnp.broadcast_to(y2c, sh), jnp.broadcast_to(y2r, sh))
        inter = jnp.maximum(xx2 - xx1, 0.0) * jnp.maximum(yy2 - yy1, 0.0)
        union = (jnp.broadcast_to(area_c, sh) + jnp.broadcast_to(area_r, sh)
                 - inter)
        iou_scr[c] = inter / jnp.maximum(union, 1e-8)
        keeps0.append(jnp.where(vals > _SCORE_THRESHOLD, 1.0, 0.0))

    def nms_step(t, keeps):
        i = 2 * t
        keep_a, keep_b = keeps
        row_a0 = iou_scr[0, pl.ds(i, 1), :]
        row_a1 = iou_scr[0, pl.ds(i + 1, 1), :]
        row_b0 = iou_scr[1, pl.ds(i, 1), :]
        row_b1 = iou_scr[1, pl.ds(i + 1, 1), :]
        e_a0 = jnp.max(jnp.where(li512 == i, keep_a, 0.0)) > 0.5
        ka1 = jnp.max(jnp.where(li512 == i + 1, keep_a, 0.0)) > 0.5
        iou_a01 = jnp.max(jnp.where(li512 == i + 1, row_a0, 0.0))
        e_b0 = jnp.max(jnp.where(li512 == i, keep_b, 0.0)) > 0.5
        kb1 = jnp.max(jnp.where(li512 == i + 1, keep_b, 0.0)) > 0.5
        iou_b01 = jnp.max(jnp.where(li512 == i + 1, row_b0, 0.0))
        e_a1 = ka1 & jnp.logical_not(e_a0 & (iou_a01 > _NMS_THRESHOLD))
        e_b1 = kb1 & jnp.logical_not(e_b0 & (iou_b01 > _NMS_THRESHOLD))
        sup_a = ((row_a0 > _NMS_THRESHOLD) & (li512 > i) & e_a0) | \
                ((row_a1 > _NMS_THRESHOLD) & (li512 > i + 1) & e_a1)
        sup_b = ((row_b0 > _NMS_THRESHOLD) & (li512 > i) & e_b0) | \
                ((row_b1 > _NMS_THRESHOLD) & (li512 > i + 1) & e_b1)
        return (jnp.where(sup_a, 0.0, keep_a),
                jnp.where(sup_b, 0.0, keep_b))

    keep_a, keep_b = jax.lax.fori_loop(0, _K // 2, nms_step,
                                       (keeps0[0], keeps0[1]))

    scores_out_ref[0] = jnp.where(keep_a > 0.5, valss[0], -1.0)
    scores_out_ref[1] = jnp.where(keep_b > 0.5, valss[1], -1.0)
    idx_out_ref[0] = idxrs[0].astype(jnp.int32)
    idx_out_ref[1] = idxrs[1].astype(jnp.int32)


def _phase2_body(sc_ref, ix_ref, boxes_ref, rot_ref, trans_ref,
                 b_out, s_out, l_out, r_out, t_out, y_scr):
    y_scr[...] = sc_ref[...]
    li = jax.lax.broadcasted_iota(jnp.int32, (1, 128), 1)
    flatiota = jax.lax.broadcasted_iota(jnp.int32, (32, 128), 0) * 128 + \
        jax.lax.broadcasted_iota(jnp.int32, (32, 128), 1)

    def pick(t, carry):
        y = y_scr[...]
        m = jnp.max(y)
        p = jnp.min(jnp.where(y == m, flatiota, _BIGI))
        pr = p >> 7
        pln = p & 127
        yrow = y_scr[pl.ds(pr, 1), :]
        y_scr[pl.ds(pr, 1), :] = jnp.where(li == pln, _NEG, yrow)
        cls = p >> 9
        f = jnp.sum(jnp.where(li == pln, ix_ref[pl.ds(pr, 1), :], 0.0)
                    ).astype(jnp.int32)
        valid = m > -0.5
        rr = f >> 7
        rl = f & 127
        lane_m = (li == rl)
        b0 = jnp.sum(jnp.where(lane_m, boxes_ref[0, pl.ds(rr, 1), :], 0.0))
        b1 = jnp.sum(jnp.where(lane_m, boxes_ref[1, pl.ds(rr, 1), :], 0.0))
        b2 = jnp.sum(jnp.where(lane_m, boxes_ref[2, pl.ds(rr, 1), :], 0.0))
        b3 = jnp.sum(jnp.where(lane_m, boxes_ref[3, pl.ds(rr, 1), :], 0.0))
        q0 = jnp.sum(jnp.where(lane_m, rot_ref[0, pl.ds(rr, 1), :], 0.0))
        q1 = jnp.sum(jnp.where(lane_m, rot_ref[1, pl.ds(rr, 1), :], 0.0))
        q2 = jnp.sum(jnp.where(lane_m, rot_ref[2, pl.ds(rr, 1), :], 0.0))
        u0 = jnp.sum(jnp.where(lane_m, trans_ref[0, pl.ds(rr, 1), :], 0.0))
        u1 = jnp.sum(jnp.where(lane_m, trans_ref[1, pl.ds(rr, 1), :], 0.0))
        u2 = jnp.sum(jnp.where(lane_m, trans_ref[2, pl.ds(rr, 1), :], 0.0))
        bvec = jnp.where(li == 0, b0,
               jnp.where(li == 1, b1,
               jnp.where(li == 2, b2,
               jnp.where(li == 3, b3, 0.0))))
        b_out[pl.ds(t, 1), :] = jnp.where(valid, bvec, -1.0)
        s_out[pl.ds(t, 1), :] = jnp.where(li == 0,
                                          jnp.where(valid, m, -1.0), 0.0)
        l_out[pl.ds(t, 1), :] = jnp.where(li == 0,
                                          jnp.where(valid, cls,
                                                    jnp.int32(-1)), 0)
        rvec = jnp.where(li == 0, q0,
               jnp.where(li == 1, q1,
               jnp.where(li == 2, q2, 0.0)))
        r_out[pl.ds(t, 1), :] = jnp.where(valid, rvec, -1.0)
        tvec = jnp.where(li == 0, u0,
               jnp.where(li == 1, u1,
               jnp.where(li == 2, u2, 0.0)))
        t_out[pl.ds(t, 1), :] = jnp.where(valid, tvec, -1.0)
        return carry

    jax.lax.fori_loop(0, _MAX_DET, pick, 0)


def _pad_cols(a_t, nrows):
    d = a_t.shape[0]
    out = jnp.zeros((d, _NPAD), a_t.dtype).at[:, :_N].set(a_t)
    return out.reshape(d, nrows, 128)


def kernel(boxes, classification, rotation, translation):
    clsP = jnp.full((_NUM_CLASSES, _NPAD), _NEG, jnp.float32)
    clsP = clsP.at[:, :_N].set(classification.T)
    clsP = clsP.reshape(_NUM_CLASSES, _ROWS, 128)
    boxesP = _pad_cols(boxes.T, _ROWS)
    rotP = _pad_cols(rotation.T, _ROWS)
    transP = _pad_cols(translation.T, _ROWS)

    scores_all, idx_all = pl.pallas_call(
        _phase1_body,
        grid=(_NUM_CLASSES // 2,),
        in_specs=[
            pl.BlockSpec((2, _ROWS, 128), lambda c: (c, 0, 0)),
            pl.BlockSpec((4, _ROWS, 128), lambda c: (0, 0, 0)),
        ],
        out_specs=[
            pl.BlockSpec((2, 1, _KPAD), lambda c: (c, 0, 0)),
            pl.BlockSpec((2, 1, _KPAD), lambda c: (c, 0, 0)),
        ],
        out_shape=[
            jax.ShapeDtypeStruct((_NUM_CLASSES, 1, _KPAD), jnp.float32),
            jax.ShapeDtypeStruct((_NUM_CLASSES, 1, _KPAD), jnp.int32),
        ],
        scratch_shapes=[
            pltpu.VMEM((2, _KPAD, 128), jnp.float32),
            pltpu.VMEM((2, _KPAD, _KPAD), jnp.float32),
        ],
    )(clsP, boxesP)

    sc = scores_all.reshape(_NUM_CLASSES * _KPAD // 128, 128)
    ix = idx_all.reshape(_NUM_CLASSES * _KPAD // 128, 128).astype(jnp.float32)

    b, s, l, r, t = pl.pallas_call(
        _phase2_body,
        out_shape=[
            jax.ShapeDtypeStruct((104, 128), jnp.float32),
            jax.ShapeDtypeStruct((104, 128), jnp.float32),
            jax.ShapeDtypeStruct((104, 128), jnp.int32),
            jax.ShapeDtypeStruct((104, 128), jnp.float32),
            jax.ShapeDtypeStruct((104, 128), jnp.float32),
        ],
        scratch_shapes=[pltpu.VMEM((32, 128), jnp.float32)],
    )(sc, ix, boxesP, rotP, transP)

    boxes_out = b[:_MAX_DET, :4]
    scores_out = s[:_MAX_DET, 0]
    labels_out = l[:_MAX_DET, 0].astype(jnp.int64)
    rotation_out = r[:_MAX_DET, :3]
    translation_out = t[:_MAX_DET, :3]
    return (boxes_out, scores_out, labels_out, rotation_out, translation_out)
